# baseline (speedup 1.0000x reference)
import functools

import jax
import jax.numpy as jnp
from jax import lax
from jax.experimental import pallas as pl
from jax.experimental.pallas import tpu as pltpu
from jax.experimental.pallas import tpu_sc as plsc

_L = 16
_NC = 2
_NS = 16
_NW = _NC * _NS
_CHUNK = 16384
_NBINS = 9
_IL = 16

_GATHER_DNUMS = lax.GatherDimensionNumbers(
    offset_dims=(), collapsed_slice_dims=(0,), start_index_map=(0,))


def _dyn_gather(vals, idx):
    return lax.gather(vals, idx[:, None], _GATHER_DNUMS, slice_sizes=(1,),
                      mode=lax.GatherScatterMode.PROMISE_IN_BOUNDS)


def _make_sc_digitize(n):
    assert n % (_NW * _CHUNK) == 0
    epw = n // _NW
    chunks = epw // _CHUNK
    assert chunks >= 4 and chunks % 2 == 0

    mesh = plsc.VectorSubcoreMesh(
        core_axis_name="c", subcore_axis_name="s",
        num_cores=_NC, num_subcores=_NS)

    @functools.partial(
        pl.kernel,
        out_type=jax.ShapeDtypeStruct((n,), jnp.int32),
        mesh=mesh,
        scratch_types=[
            pltpu.VMEM((_L,), jnp.float32),
            pltpu.VMEM((_CHUNK,), jnp.float32),
            pltpu.VMEM((_CHUNK,), jnp.float32),
            pltpu.VMEM((_CHUNK,), jnp.int32),
            pltpu.VMEM((_CHUNK,), jnp.int32),
            pltpu.SemaphoreType.DMA,
            pltpu.SemaphoreType.DMA,
            pltpu.SemaphoreType.DMA,
            pltpu.SemaphoreType.DMA,
        ],
    )
    def sc_digitize(x_hbm, binsp_hbm, out_hbm, binsp_v,
                    in0, in1, ot0, ot1, si0, si1, so0, so1):
        ins, outs = (in0, in1), (ot0, ot1)
        isems, osems = (si0, si1), (so0, so1)
        wid = lax.axis_index("s") * _NC + lax.axis_index("c")
        pltpu.sync_copy(binsp_hbm, binsp_v)
        ball = binsp_v[...]
        idx7 = jnp.full((_L,), 7, jnp.int32)
        b7 = _dyn_gather(ball, idx7)
        lanes = lax.iota(jnp.int32, _L)
        t2 = _dyn_gather(ball, lanes | 3)
        t1 = _dyn_gather(ball, lanes | 1)
        base0 = wid * epw

        def src(g):
            return x_hbm.at[pl.ds(base0 + g * _CHUNK, _CHUNK)]

        def dst(g):
            return out_hbm.at[pl.ds(base0 + g * _CHUNK, _CHUNK)]

        def in_start(g, b):
            pltpu.async_copy(src(g), ins[b], isems[b])

        def in_wait(g, b):
            pltpu.make_async_copy(src(g), ins[b], isems[b]).wait()

        def out_start(g, b):
            pltpu.async_copy(outs[b], dst(g), osems[b])

        def out_wait(g, b):
            pltpu.make_async_copy(outs[b], dst(g), osems[b]).wait()

        def compute(b):
            in_v, out_v = ins[b], outs[b]

            @plsc.parallel_loop(0, _CHUNK // (_L * _IL), 1, unroll=2)
            def vec_body(i):
                sls = [pl.ds((i * _IL + k) * _L, _L) for k in range(_IL)]
                xs = [in_v[sl] for sl in sls]
                cs = [jnp.where(x >= b7, 8, 0) for x in xs]
                vs = [_dyn_gather(t2, c0) for c0 in cs]
                cs = [c0 | jnp.where(x >= v, 4, 0)
                      for c0, v, x in zip(cs, vs, xs)]
                vs = [_dyn_gather(t1, c0) for c0 in cs]
                cs = [c0 | jnp.where(x >= v, 2, 0)
                      for c0, v, x in zip(cs, vs, xs)]
                vs = [_dyn_gather(ball, c0) for c0 in cs]
                cs = [c0 | jnp.where(x >= v, 1, 0)
                      for c0, v, x in zip(cs, vs, xs)]
                for sl, c0 in zip(sls, cs):
                    out_v[sl] = c0

        in_start(0, 0)
        in_start(1, 1)
        for g in (0, 1):
            in_wait(g, g)
            compute(g)
            out_start(g, g)
            in_start(g + 2, g)

        def steady(i, carry):
            for b in (0, 1):
                g = 2 * i + b
                in_wait(g, b)
                out_wait(g - 2, b)
                compute(b)
                out_start(g, b)
                in_start(g + 2, b)
            return carry

        lax.fori_loop(1, chunks // 2 - 1, steady, 0)

        for b in (0, 1):
            g = chunks - 2 + b
            in_wait(g, b)
            out_wait(g - 2, b)
            compute(b)
            out_start(g, b)
        out_wait(chunks - 2, 0)
        out_wait(chunks - 1, 1)

    return sc_digitize


def kernel(input, OH_bins):
    n = input.shape[0]
    binsp = jnp.concatenate(
        [OH_bins, jnp.full((_L - _NBINS,), jnp.inf, jnp.float32)])
    return _make_sc_digitize(n)(input, binsp)


# speedup vs baseline: 2.8188x; 2.8188x over previous
import functools

import jax
import jax.numpy as jnp
from jax import lax
from jax.experimental import pallas as pl
from jax.experimental.pallas import tpu as pltpu
from jax.experimental.pallas import tpu_sc as plsc

_L = 16
_NC = 2
_NS = 16
_NW = _NC * _NS
_CHUNK = 16384
_NBINS = 9
_IL = 8

_GATHER_DNUMS = lax.GatherDimensionNumbers(
    offset_dims=(), collapsed_slice_dims=(0,), start_index_map=(0,))


def _dyn_gather(vals, idx):
    return lax.gather(vals, idx[:, None], _GATHER_DNUMS, slice_sizes=(1,),
                      mode=lax.GatherScatterMode.PROMISE_IN_BOUNDS)


def _make_sc_digitize(n):
    assert n % (_NW * _CHUNK) == 0
    epw = n // _NW
    chunks = epw // _CHUNK
    assert chunks >= 4 and chunks % 2 == 0

    mesh = plsc.VectorSubcoreMesh(
        core_axis_name="c", subcore_axis_name="s",
        num_cores=_NC, num_subcores=_NS)

    @functools.partial(
        pl.kernel,
        out_type=jax.ShapeDtypeStruct((n,), jnp.int32),
        mesh=mesh,
        scratch_types=[
            pltpu.VMEM((_L,), jnp.float32),
            pltpu.VMEM((_CHUNK,), jnp.float32),
            pltpu.VMEM((_CHUNK,), jnp.float32),
            pltpu.VMEM((_CHUNK,), jnp.int32),
            pltpu.VMEM((_CHUNK,), jnp.int32),
            pltpu.SemaphoreType.DMA,
            pltpu.SemaphoreType.DMA,
            pltpu.SemaphoreType.DMA,
            pltpu.SemaphoreType.DMA,
        ],
    )
    def sc_digitize(x_hbm, binsp_hbm, out_hbm, binsp_v,
                    in0, in1, ot0, ot1, si0, si1, so0, so1):
        ins, outs = (in0, in1), (ot0, ot1)
        isems, osems = (si0, si1), (so0, so1)
        wid = lax.axis_index("s") * _NC + lax.axis_index("c")
        pltpu.sync_copy(binsp_hbm, binsp_v)
        ball = binsp_v[...]
        idx7 = jnp.full((_L,), 7, jnp.int32)
        b7 = _dyn_gather(ball, idx7)
        lanes = lax.iota(jnp.int32, _L)
        t2 = _dyn_gather(ball, lanes | 3)
        t1 = _dyn_gather(ball, lanes | 1)
        base0 = wid * epw

        def src(g):
            return x_hbm.at[pl.ds(base0 + g * _CHUNK, _CHUNK)]

        def dst(g):
            return out_hbm.at[pl.ds(base0 + g * _CHUNK, _CHUNK)]

        def in_start(g, b):
            pltpu.async_copy(src(g), ins[b], isems[b])

        def in_wait(g, b):
            pltpu.make_async_copy(src(g), ins[b], isems[b]).wait()

        def out_start(g, b):
            pltpu.async_copy(outs[b], dst(g), osems[b])

        def out_wait(g, b):
            pltpu.make_async_copy(outs[b], dst(g), osems[b]).wait()

        def compute(b):
            in_v, out_v = ins[b], outs[b]

            @plsc.parallel_loop(0, _CHUNK // (_L * _IL), 1, unroll=2)
            def vec_body(i):
                sls = [pl.ds((i * _IL + k) * _L, _L) for k in range(_IL)]
                xs = [in_v[sl] for sl in sls]
                cs = [jnp.where(x >= b7, 8, 0) for x in xs]
                vs = [_dyn_gather(t2, c0) for c0 in cs]
                cs = [c0 | jnp.where(x >= v, 4, 0)
                      for c0, v, x in zip(cs, vs, xs)]
                vs = [_dyn_gather(t1, c0) for c0 in cs]
                cs = [c0 | jnp.where(x >= v, 2, 0)
                      for c0, v, x in zip(cs, vs, xs)]
                vs = [_dyn_gather(ball, c0) for c0 in cs]
                cs = [c0 | jnp.where(x >= v, 1, 0)
                      for c0, v, x in zip(cs, vs, xs)]
                for sl, c0 in zip(sls, cs):
                    out_v[sl] = c0

        in_start(0, 0)
        in_start(1, 1)
        for g in (0, 1):
            in_wait(g, g)
            compute(g)
            out_start(g, g)
            in_start(g + 2, g)

        def steady(i, carry):
            for b in (0, 1):
                g = 2 * i + b
                in_wait(g, b)
                out_wait(g - 2, b)
                compute(b)
                out_start(g, b)
                in_start(g + 2, b)
            return carry

        lax.fori_loop(1, chunks // 2 - 1, steady, 0)

        for b in (0, 1):
            g = chunks - 2 + b
            in_wait(g, b)
            out_wait(g - 2, b)
            compute(b)
            out_start(g, b)
        out_wait(chunks - 2, 0)
        out_wait(chunks - 1, 1)

    return sc_digitize


def kernel(input, OH_bins):
    n = input.shape[0]
    binsp = jnp.concatenate(
        [OH_bins, jnp.full((_L - _NBINS,), jnp.inf, jnp.float32)])
    return _make_sc_digitize(n)(input, binsp)
